# baseline (speedup 1.0000x reference)
import functools

import jax
import jax.numpy as jnp
from jax import lax
from jax.experimental import pallas as pl
from jax.experimental.pallas import tpu as pltpu
from jax.experimental.pallas import tpu_sc as plsc

NUM_CORES = 2
NUM_SUBCORES = 16
NUM_WORKERS = NUM_CORES * NUM_SUBCORES
LANES = 16
FACTORS = 64
B = 16384
V = 1000000
CW = 128
NCH = V // CW + 1
TAIL_BASE = (V // CW) * CW
CPW = -(-NCH // NUM_WORKERS)
RING = 64
OPAD = B + 16


def _phase1_body(tt_u, tt_i, tl_u, tl_i,
                 ord_u, val_u, st_u, ord_i, val_i, st_i,
                 ord_j, val_j, st_j,
                 stg_u, stg_i, stg_j,
                 stA_v, stB_v, stC_v, ordA, valA, ordB, valB,
                 chunk3, xfer, csem0, csem1, xsem):
    wid = lax.axis_index("s") * NUM_CORES + lax.axis_index("c")
    c_lo = wid * CPW
    c_hi = jnp.minimum(c_lo + CPW, NCH)
    c_al = pl.multiple_of((c_lo // 8) * 8, 8)
    iota16 = lax.iota(jnp.int32, LANES)

    pltpu.sync_copy(st_u.at[pl.ds(c_al, 272)], stA_v)
    pltpu.sync_copy(st_i.at[pl.ds(c_al, 272)], stB_v)
    pltpu.sync_copy(st_j.at[pl.ds(c_al, 272)], stC_v)

    def issue_chunk(c2, tt, tl):
        par = lax.rem(c2 - c_lo, 2)

        @pl.when(jnp.logical_and(c2 < c_hi, par == 0))
        def _():
            @pl.when(c2 < NCH - 1)
            def _():
                pltpu.async_copy(tt.at[:, pl.ds(c2 * CW, CW)],
                                 chunk3.at[0], csem0)

            @pl.when(c2 == NCH - 1)
            def _():
                pltpu.async_copy(tl.at[:, :], chunk3.at[0], csem0)

        @pl.when(jnp.logical_and(c2 < c_hi, par == 1))
        def _():
            @pl.when(c2 < NCH - 1)
            def _():
                pltpu.async_copy(tt.at[:, pl.ds(c2 * CW, CW)],
                                 chunk3.at[1], csem1)

            @pl.when(c2 == NCH - 1)
            def _():
                pltpu.async_copy(tl.at[:, :], chunk3.at[1], csem1)

    def wait_chunk(c):
        par = lax.rem(c - c_lo, 2)

        @pl.when(par == 0)
        def _():
            pltpu.make_async_copy(tl_u.at[:, :], chunk3.at[0], csem0).wait()

        @pl.when(par == 1)
        def _():
            pltpu.make_async_copy(tl_u.at[:, :], chunk3.at[1], csem1).wait()

    def do_hits(c, par, st_v, ord_v, val_v, stg, nx0):
        cv = jnp.full((LANES,), c - c_al, jnp.int32)
        s0 = plsc.load_gather(st_v, [cv])[0]
        s1 = plsc.load_gather(st_v, [cv + 1])[0]
        cbase = c * CW

        def hit(p, nx):
            pv = jnp.full((LANES,), p, jnp.int32)
            bpos = plsc.load_gather(ord_v, [pv])[0]
            col_v = plsc.load_gather(val_v, [pv]) - cbase
            slot = lax.rem(nx, RING)
            parv = jnp.full((LANES,), par, jnp.int32)
            for t in range(4):
                xfer[pl.ds(slot * FACTORS + t * LANES, LANES)] = (
                    plsc.load_gather(chunk3, [parv, t * LANES + iota16,
                                              col_v]))
            pltpu.async_copy(xfer.at[pl.ds(slot * FACTORS, FACTORS)],
                             stg.at[pl.ds(bpos * FACTORS, FACTORS)], xsem)

            @pl.when(slot == RING - 1)
            def _():
                for _k in range(RING):
                    pltpu.make_async_copy(
                        xfer.at[pl.ds(0, FACTORS)],
                        stg.at[pl.ds(0, FACTORS)], xsem).wait()

            return nx + 1

        return lax.fori_loop(s0, s1, hit, nx0)

    def run_pass(tt, tl, csrs, nx0):
        issue_chunk(c_lo, tt, tl)
        issue_chunk(c_lo + 1, tt, tl)

        def cbody(c, nx):
            par = lax.rem(c - c_lo, 2)
            wait_chunk(c)
            for (st_v, ord_v, val_v, stg) in csrs:
                nx = do_hits(c, par, st_v, ord_v, val_v, stg, nx)
            issue_chunk(c + 2, tt, tl)
            return nx

        return lax.fori_loop(c_lo, c_hi, cbody, nx0)

    pltpu.sync_copy(ord_u, ordA)
    pltpu.sync_copy(val_u, valA)
    nx = run_pass(tt_u, tl_u, [(stA_v, ordA, valA, stg_u)], 0)

    pltpu.sync_copy(ord_i, ordA)
    pltpu.sync_copy(val_i, valA)
    pltpu.sync_copy(ord_j, ordB)
    pltpu.sync_copy(val_j, valB)
    nx = run_pass(tt_i, tl_i, [(stB_v, ordA, valA, stg_i),
                               (stC_v, ordB, valB, stg_j)], nx)

    rem = lax.rem(nx, RING)
    for k in range(RING):
        @pl.when(k < rem)
        def _():
            pltpu.make_async_copy(xfer.at[pl.ds(0, FACTORS)],
                                  stg_j.at[pl.ds(0, FACTORS)], xsem).wait()


def _phase2_body(stg_u, stg_i, stg_j, b1, iidx, jidx, out_hbm,
                 u_buf, i_buf, j_buf, ii_v, jj_v, bi_v, bj_v, out_v, sem):
    b_per_w = ii_v.shape[0]
    wid = lax.axis_index("s") * NUM_CORES + lax.axis_index("c")
    base = wid * b_per_w

    pltpu.sync_copy(stg_u.at[pl.ds(base * FACTORS, b_per_w * FACTORS)], u_buf)
    pltpu.sync_copy(stg_i.at[pl.ds(base * FACTORS, b_per_w * FACTORS)], i_buf)
    pltpu.sync_copy(stg_j.at[pl.ds(base * FACTORS, b_per_w * FACTORS)], j_buf)
    pltpu.sync_copy(iidx.at[pl.ds(base, b_per_w)], ii_v)
    pltpu.sync_copy(jidx.at[pl.ds(base, b_per_w)], jj_v)

    copies = []
    for k in range(b_per_w // 128):
        sl = pl.ds(k * 128, 128)
        copies.append(pltpu.async_copy(b1.at[ii_v.at[sl]], bi_v.at[sl], sem))
        copies.append(pltpu.async_copy(b1.at[jj_v.at[sl]], bj_v.at[sl], sem))
    for c in copies:
        c.wait()

    iota16 = lax.iota(jnp.int32, LANES)

    def group_body(g, _):
        rbase = g * LANES
        acc = bi_v[pl.ds(rbase, LANES)] - bj_v[pl.ds(rbase, LANES)]
        flat0 = (rbase + iota16) * FACTORS
        for f in range(FACTORS):
            flat = flat0 + f
            u = plsc.load_gather(u_buf, [flat])
            iv = plsc.load_gather(i_buf, [flat])
            jv = plsc.load_gather(j_buf, [flat])
            acc = acc + u * (iv - jv)
        out_v[pl.ds(rbase, LANES)] = acc
        return 0

    lax.fori_loop(0, b_per_w // LANES, group_body, 0)

    pltpu.sync_copy(out_v, out_hbm.at[pl.ds(base, b_per_w)])


def _make_phase1():
    mesh = plsc.VectorSubcoreMesh(core_axis_name="c", subcore_axis_name="s")
    return pl.kernel(
        _phase1_body,
        mesh=mesh,
        compiler_params=pltpu.CompilerParams(needs_layout_passes=False),
        out_type=(jax.ShapeDtypeStruct((B * FACTORS,), jnp.float32),
                  jax.ShapeDtypeStruct((B * FACTORS,), jnp.float32),
                  jax.ShapeDtypeStruct((B * FACTORS,), jnp.float32)),
        scratch_types=[
            pltpu.VMEM((272,), jnp.int32),
            pltpu.VMEM((272,), jnp.int32),
            pltpu.VMEM((272,), jnp.int32),
            pltpu.VMEM((OPAD,), jnp.int32),
            pltpu.VMEM((OPAD,), jnp.int32),
            pltpu.VMEM((OPAD,), jnp.int32),
            pltpu.VMEM((OPAD,), jnp.int32),
            pltpu.VMEM((2, FACTORS, CW), jnp.float32),
            pltpu.VMEM((RING * FACTORS,), jnp.float32),
            pltpu.SemaphoreType.DMA,
            pltpu.SemaphoreType.DMA,
            pltpu.SemaphoreType.DMA,
        ],
    )


def _make_phase2():
    b_per_w = B // NUM_WORKERS
    mesh = plsc.VectorSubcoreMesh(core_axis_name="c", subcore_axis_name="s")
    return pl.kernel(
        _phase2_body,
        mesh=mesh,
        compiler_params=pltpu.CompilerParams(needs_layout_passes=False),
        out_type=jax.ShapeDtypeStruct((B,), jnp.float32),
        scratch_types=[
            pltpu.VMEM((b_per_w * FACTORS,), jnp.float32),
            pltpu.VMEM((b_per_w * FACTORS,), jnp.float32),
            pltpu.VMEM((b_per_w * FACTORS,), jnp.float32),
            pltpu.VMEM((b_per_w,), jnp.int32),
            pltpu.VMEM((b_per_w,), jnp.int32),
            pltpu.VMEM((b_per_w,), jnp.float32),
            pltpu.VMEM((b_per_w,), jnp.float32),
            pltpu.VMEM((b_per_w,), jnp.float32),
            pltpu.SemaphoreType.DMA,
        ],
    )


def kernel(user, item_i, item_j, user_table, item_table, item_bias_table):
    u32 = user.astype(jnp.int32)
    i32 = item_i.astype(jnp.int32)
    j32 = item_j.astype(jnp.int32)

    tt_u = user_table.T
    tt_i = item_table.T
    tl_u = jnp.pad(user_table[TAIL_BASE:].T, ((0, 0), (0, CW - (V - TAIL_BASE))))
    tl_i = jnp.pad(item_table[TAIL_BASE:].T, ((0, 0), (0, CW - (V - TAIL_BASE))))
    b1 = item_bias_table.reshape(-1)

    iota = jnp.arange(B, dtype=jnp.int32)
    edges = jnp.arange(NCH + 1, dtype=jnp.int32) * CW

    def csr(a):
        vals, order = lax.sort_key_val(a, iota)
        starts = jnp.searchsorted(vals, edges).astype(jnp.int32)
        return (jnp.pad(order, (0, OPAD - B)), jnp.pad(vals, (0, OPAD - B)),
                jnp.pad(starts, (0, 64)))

    ou, vu, su = csr(u32)
    oi, vi, si = csr(i32)
    oj, vj, sj = csr(j32)

    stg_u, stg_i, stg_j = _make_phase1()(
        tt_u, tt_i, tl_u, tl_i, ou, vu, su, oi, vi, si, oj, vj, sj)
    return _make_phase2()(stg_u, stg_i, stg_j, b1, i32, j32)


# speedup vs baseline: 2.4707x; 2.4707x over previous
import functools

import jax
import jax.numpy as jnp
from jax import lax
from jax.experimental import pallas as pl
from jax.experimental.pallas import tpu as pltpu
from jax.experimental.pallas import tpu_sc as plsc

NUM_CORES = 2
NUM_SUBCORES = 16
NUM_WORKERS = NUM_CORES * NUM_SUBCORES
LANES = 16
FACTORS = 64
B = 16384
V = 1000000
CW = 128
NCH = V // CW + 1
TAIL_BASE = (V // CW) * CW
CPW = -(-NCH // NUM_WORKERS)
RING = 64
LPAD = B + 32
SENTINEL = jnp.int32(2**30)


def _phase1_body(tt_u, tt_i, tl_u, tl_i, uidx, iidx, jidx,
                 stg_u, stg_i, stg_j,
                 idx_buf, listU, listI, listJ, tmp16,
                 chunk3, xfer, csem0, csem1, xsem):
    wid = lax.axis_index("s") * NUM_CORES + lax.axis_index("c")
    c_lo = wid * CPW
    c_hi = jnp.minimum(c_lo + CPW, NCH)
    base_u = c_lo * CW
    hi_u = c_hi * CW
    iota16 = lax.iota(jnp.int32, LANES)
    mask0 = iota16 == 0

    def popcnt(m):
        return plsc.all_reduce_population_count(m)[0]

    def ffs(m):
        return plsc.all_reduce_ffs(m)[0]

    def prefilter(src_hbm, list_ref):
        pltpu.sync_copy(src_hbm, idx_buf)

        def pf(i, L):
            v = idx_buf[pl.ds(i * LANES, LANES)]
            m = jnp.logical_and(v >= base_u, v < hi_u)
            packed = jnp.left_shift(v - base_u, 14) | (i * LANES + iota16)
            tmp16[pl.ds(0, LANES)] = packed

            def put(_, st):
                L2, mm = st
                lane = ffs(mm)
                pk = plsc.load_gather(tmp16, [jnp.full((LANES,), lane,
                                                       jnp.int32)])
                plsc.store_scatter(list_ref,
                                   [jnp.full((LANES,), L2, jnp.int32)],
                                   pk, mask=mask0)
                return L2 + 1, jnp.logical_and(mm, iota16 != lane)

            L, _ = lax.fori_loop(0, popcnt(m), put, (L, m))
            return L

        L = lax.fori_loop(0, B // LANES, pf, 0)
        plsc.store_scatter(list_ref, [L + iota16],
                           jnp.full((LANES,), SENTINEL, jnp.int32))
        return L

    def issue_chunk(c2, tt, tl):
        par = lax.rem(c2 - c_lo, 2)

        @pl.when(jnp.logical_and(c2 < c_hi, par == 0))
        def _():
            @pl.when(c2 < NCH - 1)
            def _():
                pltpu.async_copy(tt.at[:, pl.ds(c2 * CW, CW)],
                                 chunk3.at[0], csem0)

            @pl.when(c2 == NCH - 1)
            def _():
                pltpu.async_copy(tl.at[:, :], chunk3.at[0], csem0)

        @pl.when(jnp.logical_and(c2 < c_hi, par == 1))
        def _():
            @pl.when(c2 < NCH - 1)
            def _():
                pltpu.async_copy(tt.at[:, pl.ds(c2 * CW, CW)],
                                 chunk3.at[1], csem1)

            @pl.when(c2 == NCH - 1)
            def _():
                pltpu.async_copy(tl.at[:, :], chunk3.at[1], csem1)

    def wait_chunk(c):
        par = lax.rem(c - c_lo, 2)

        @pl.when(par == 0)
        def _():
            pltpu.make_async_copy(tl_u.at[:, :], chunk3.at[0], csem0).wait()

        @pl.when(par == 1)
        def _():
            pltpu.make_async_copy(tl_u.at[:, :], chunk3.at[1], csem1).wait()

    def do_hits(c, par, list_ref, nvec, stg, nx0):
        lo_c = jnp.left_shift((c - c_lo) * CW, 14)
        hi_c = jnp.left_shift((c - c_lo) * CW + CW, 14)
        parv = jnp.full((LANES,), par, jnp.int32)

        def sb(i, nx):
            pv = list_ref[pl.ds(i * LANES, LANES)]
            m = jnp.logical_and(pv >= lo_c, pv < hi_c)

            def hh(_, st):
                nx2, mm = st
                lane = ffs(mm)
                q = i * LANES + lane
                pj = plsc.load_gather(
                    list_ref, [jnp.full((LANES,), q, jnp.int32)])[0]
                col = (pj >> 14) - (c - c_lo) * CW
                bpos = pj & (2**14 - 1)
                col_v = jnp.full((LANES,), col, jnp.int32)
                slot = lax.rem(nx2, RING)
                for t in range(4):
                    xfer[pl.ds(slot * FACTORS + t * LANES, LANES)] = (
                        plsc.load_gather(
                            chunk3, [parv, t * LANES + iota16, col_v]))
                pltpu.async_copy(
                    xfer.at[pl.ds(slot * FACTORS, FACTORS)],
                    stg.at[pl.ds(bpos * FACTORS, FACTORS)], xsem)

                @pl.when(slot == RING - 1)
                def _():
                    for _k in range(RING):
                        pltpu.make_async_copy(
                            xfer.at[pl.ds(0, FACTORS)],
                            stg.at[pl.ds(0, FACTORS)], xsem).wait()

                return nx2 + 1, jnp.logical_and(mm, iota16 != lane)

            nx, _ = lax.fori_loop(0, popcnt(m), hh, (nx, m))
            return nx

        return lax.fori_loop(0, nvec, sb, nx0)

    def run_pass(tt, tl, csrs, nx0):
        issue_chunk(c_lo, tt, tl)
        issue_chunk(c_lo + 1, tt, tl)

        def cbody(c, nx):
            par = lax.rem(c - c_lo, 2)
            wait_chunk(c)
            for (list_ref, nvec, stg) in csrs:
                nx = do_hits(c, par, list_ref, nvec, stg, nx)
            issue_chunk(c + 2, tt, tl)
            return nx

        return lax.fori_loop(c_lo, c_hi, cbody, nx0)

    lu = prefilter(uidx, listU)
    li = prefilter(iidx, listI)
    lj = prefilter(jidx, listJ)
    nvu = (lu + LANES) // LANES
    nvi = (li + LANES) // LANES
    nvj = (lj + LANES) // LANES

    nx = run_pass(tt_u, tl_u, [(listU, nvu, stg_u)], 0)
    nx = run_pass(tt_i, tl_i, [(listI, nvi, stg_i),
                               (listJ, nvj, stg_j)], nx)

    rem = lax.rem(nx, RING)
    for k in range(RING):
        @pl.when(k < rem)
        def _():
            pltpu.make_async_copy(xfer.at[pl.ds(0, FACTORS)],
                                  stg_j.at[pl.ds(0, FACTORS)], xsem).wait()


def _phase2_body(stg_u, stg_i, stg_j, b1, iidx, jidx, out_hbm,
                 u_buf, i_buf, j_buf, ii_v, jj_v, bi_v, bj_v, out_v, sem):
    b_per_w = ii_v.shape[0]
    wid = lax.axis_index("s") * NUM_CORES + lax.axis_index("c")
    base = wid * b_per_w

    pltpu.sync_copy(stg_u.at[pl.ds(base * FACTORS, b_per_w * FACTORS)], u_buf)
    pltpu.sync_copy(stg_i.at[pl.ds(base * FACTORS, b_per_w * FACTORS)], i_buf)
    pltpu.sync_copy(stg_j.at[pl.ds(base * FACTORS, b_per_w * FACTORS)], j_buf)
    pltpu.sync_copy(iidx.at[pl.ds(base, b_per_w)], ii_v)
    pltpu.sync_copy(jidx.at[pl.ds(base, b_per_w)], jj_v)

    copies = []
    for k in range(b_per_w // 128):
        sl = pl.ds(k * 128, 128)
        copies.append(pltpu.async_copy(b1.at[ii_v.at[sl]], bi_v.at[sl], sem))
        copies.append(pltpu.async_copy(b1.at[jj_v.at[sl]], bj_v.at[sl], sem))
    for c in copies:
        c.wait()

    iota16 = lax.iota(jnp.int32, LANES)

    def group_body(g, _):
        rbase = g * LANES
        acc = bi_v[pl.ds(rbase, LANES)] - bj_v[pl.ds(rbase, LANES)]
        flat0 = (rbase + iota16) * FACTORS
        for f in range(FACTORS):
            flat = flat0 + f
            u = plsc.load_gather(u_buf, [flat])
            iv = plsc.load_gather(i_buf, [flat])
            jv = plsc.load_gather(j_buf, [flat])
            acc = acc + u * (iv - jv)
        out_v[pl.ds(rbase, LANES)] = acc
        return 0

    lax.fori_loop(0, b_per_w // LANES, group_body, 0)

    pltpu.sync_copy(out_v, out_hbm.at[pl.ds(base, b_per_w)])


def _make_phase1():
    mesh = plsc.VectorSubcoreMesh(core_axis_name="c", subcore_axis_name="s")
    return pl.kernel(
        _phase1_body,
        mesh=mesh,
        compiler_params=pltpu.CompilerParams(needs_layout_passes=False),
        out_type=(jax.ShapeDtypeStruct((B * FACTORS,), jnp.float32),
                  jax.ShapeDtypeStruct((B * FACTORS,), jnp.float32),
                  jax.ShapeDtypeStruct((B * FACTORS,), jnp.float32)),
        scratch_types=[
            pltpu.VMEM((B,), jnp.int32),
            pltpu.VMEM((LPAD,), jnp.int32),
            pltpu.VMEM((LPAD,), jnp.int32),
            pltpu.VMEM((LPAD,), jnp.int32),
            pltpu.VMEM((LANES,), jnp.int32),
            pltpu.VMEM((2, FACTORS, CW), jnp.float32),
            pltpu.VMEM((RING * FACTORS,), jnp.float32),
            pltpu.SemaphoreType.DMA,
            pltpu.SemaphoreType.DMA,
            pltpu.SemaphoreType.DMA,
        ],
    )


def _make_phase2():
    b_per_w = B // NUM_WORKERS
    mesh = plsc.VectorSubcoreMesh(core_axis_name="c", subcore_axis_name="s")
    return pl.kernel(
        _phase2_body,
        mesh=mesh,
        compiler_params=pltpu.CompilerParams(needs_layout_passes=False),
        out_type=jax.ShapeDtypeStruct((B,), jnp.float32),
        scratch_types=[
            pltpu.VMEM((b_per_w * FACTORS,), jnp.float32),
            pltpu.VMEM((b_per_w * FACTORS,), jnp.float32),
            pltpu.VMEM((b_per_w * FACTORS,), jnp.float32),
            pltpu.VMEM((b_per_w,), jnp.int32),
            pltpu.VMEM((b_per_w,), jnp.int32),
            pltpu.VMEM((b_per_w,), jnp.float32),
            pltpu.VMEM((b_per_w,), jnp.float32),
            pltpu.VMEM((b_per_w,), jnp.float32),
            pltpu.SemaphoreType.DMA,
        ],
    )


def kernel(user, item_i, item_j, user_table, item_table, item_bias_table):
    u32 = user.astype(jnp.int32)
    i32 = item_i.astype(jnp.int32)
    j32 = item_j.astype(jnp.int32)

    tt_u = user_table.T
    tt_i = item_table.T
    tl_u = jnp.pad(user_table[TAIL_BASE:].T,
                   ((0, 0), (0, CW - (V - TAIL_BASE))))
    tl_i = jnp.pad(item_table[TAIL_BASE:].T,
                   ((0, 0), (0, CW - (V - TAIL_BASE))))
    b1 = item_bias_table.reshape(-1)

    stg_u, stg_i, stg_j = _make_phase1()(
        tt_u, tt_i, tl_u, tl_i, u32, i32, j32)
    return _make_phase2()(stg_u, stg_i, stg_j, b1, i32, j32)


# speedup vs baseline: 4.0948x; 1.6574x over previous
import functools

import jax
import jax.numpy as jnp
from jax import lax
from jax.experimental import pallas as pl
from jax.experimental.pallas import tpu as pltpu
from jax.experimental.pallas import tpu_sc as plsc

NUM_CORES = 2
NUM_SUBCORES = 16
NUM_WORKERS = NUM_CORES * NUM_SUBCORES
LANES = 16
FACTORS = 64
B = 16384
V = 1000000
CW = 128
NCH = V // CW + 1
TAIL_BASE = (V // CW) * CW
CPW = -(-NCH // NUM_WORKERS)
RING = 64
LPAD = B + 32
SENTINEL = jnp.int32(2**30)


def _phase1_body(tt_u, tt_i, tl_u, tl_i, uidx, iidx, jidx,
                 stg_u, stg_i, stg_j,
                 idx_buf, listU, listI, listJ, tmp16,
                 cntU, prefU, wrkU, cntI, prefI, wrkI, cntJ, prefJ, wrkJ,
                 chunk3, xfer, csem0, csem1, xsem):
    wid = lax.axis_index("s") * NUM_CORES + lax.axis_index("c")
    c_lo = wid * CPW
    c_hi = jnp.minimum(c_lo + CPW, NCH)
    base_u = c_lo * CW
    hi_u = c_hi * CW
    iota16 = lax.iota(jnp.int32, LANES)
    mask0 = iota16 == 0

    def popcnt(m):
        return plsc.all_reduce_population_count(m)[0]

    def ffs(m):
        return plsc.all_reduce_ffs(m)[0]

    def prefilter(src_hbm, list_ref, counts, prefix, wrk):
        pltpu.sync_copy(src_hbm, idx_buf)
        zeros16 = jnp.zeros((LANES,), jnp.int32)
        ones16 = jnp.ones((LANES,), jnp.int32)
        for i in range(16):
            counts[pl.ds(i * LANES, LANES)] = zeros16

        def cnt(i, _):
            v = idx_buf[pl.ds(i * LANES, LANES)]
            m = jnp.logical_and(v >= base_u, v < hi_u)
            cidl = lax.shift_right_logical(v - base_u, 7)
            plsc.addupdate_scatter(counts, [cidl], ones16, mask=m)
            return 0

        lax.fori_loop(0, B // LANES, cnt, 0)

        plsc.store_scatter(prefix, [iota16], zeros16, mask=mask0)
        running = jnp.int32(0)
        for i in range(16):
            c16 = counts[pl.ds(i * LANES, LANES)]
            cs = plsc.cumsum(c16) + running
            plsc.store_scatter(prefix, [i * LANES + 1 + iota16], cs)
            running = cs[15]
        for i in range(16):
            wrk[pl.ds(i * LANES, LANES)] = prefix[pl.ds(i * LANES, LANES)]

        def place(i, _):
            v = idx_buf[pl.ds(i * LANES, LANES)]
            m = jnp.logical_and(v >= base_u, v < hi_u)
            packed = jnp.left_shift(v - base_u, 14) | (i * LANES + iota16)
            tmp16[pl.ds(0, LANES)] = packed

            def put(_, mm):
                lane = ffs(mm)
                lv = jnp.full((LANES,), lane, jnp.int32)
                pk = plsc.load_gather(tmp16, [lv])
                cidl = lax.shift_right_logical(pk, 21)
                pos = plsc.load_gather(wrk, [cidl])
                plsc.store_scatter(list_ref, [pos], pk, mask=mask0)
                plsc.store_scatter(wrk, [cidl], pos + 1, mask=mask0)
                return jnp.logical_and(mm, iota16 != lane)

            lax.fori_loop(0, popcnt(m), put, m)
            return 0

        lax.fori_loop(0, B // LANES, place, 0)

    def issue_chunk(c2, tt, tl):
        par = lax.rem(c2 - c_lo, 2)

        @pl.when(jnp.logical_and(c2 < c_hi, par == 0))
        def _():
            @pl.when(c2 < NCH - 1)
            def _():
                pltpu.async_copy(tt.at[:, pl.ds(c2 * CW, CW)],
                                 chunk3.at[0], csem0)

            @pl.when(c2 == NCH - 1)
            def _():
                pltpu.async_copy(tl.at[:, :], chunk3.at[0], csem0)

        @pl.when(jnp.logical_and(c2 < c_hi, par == 1))
        def _():
            @pl.when(c2 < NCH - 1)
            def _():
                pltpu.async_copy(tt.at[:, pl.ds(c2 * CW, CW)],
                                 chunk3.at[1], csem1)

            @pl.when(c2 == NCH - 1)
            def _():
                pltpu.async_copy(tl.at[:, :], chunk3.at[1], csem1)

    def wait_chunk(c):
        par = lax.rem(c - c_lo, 2)

        @pl.when(par == 0)
        def _():
            pltpu.make_async_copy(tl_u.at[:, :], chunk3.at[0], csem0).wait()

        @pl.when(par == 1)
        def _():
            pltpu.make_async_copy(tl_u.at[:, :], chunk3.at[1], csem1).wait()

    def do_hits(c, par, list_ref, prefix, stg, nx0):
        cv = jnp.full((LANES,), c - c_lo, jnp.int32)
        s0 = plsc.load_gather(prefix, [cv])[0]
        s1 = plsc.load_gather(prefix, [cv + 1])[0]
        parv = jnp.full((LANES,), par, jnp.int32)

        def hh(q, nx2):
            pj = plsc.load_gather(
                list_ref, [jnp.full((LANES,), q, jnp.int32)])[0]
            col = (pj >> 14) - (c - c_lo) * CW
            bpos = pj & (2**14 - 1)
            col_v = jnp.full((LANES,), col, jnp.int32)
            slot = lax.rem(nx2, RING)
            for t in range(4):
                xfer[pl.ds(slot * FACTORS + t * LANES, LANES)] = (
                    plsc.load_gather(
                        chunk3, [parv, t * LANES + iota16, col_v]))
            pltpu.async_copy(
                xfer.at[pl.ds(slot * FACTORS, FACTORS)],
                stg.at[pl.ds(bpos * FACTORS, FACTORS)], xsem)

            @pl.when(slot == RING - 1)
            def _():
                for _k in range(RING):
                    pltpu.make_async_copy(
                        xfer.at[pl.ds(0, FACTORS)],
                        stg.at[pl.ds(0, FACTORS)], xsem).wait()

            return nx2 + 1

        return lax.fori_loop(s0, s1, hh, nx0)

    def run_pass(tt, tl, csrs, nx0):
        issue_chunk(c_lo, tt, tl)
        issue_chunk(c_lo + 1, tt, tl)

        def cbody(c, nx):
            par = lax.rem(c - c_lo, 2)
            wait_chunk(c)
            for (list_ref, prefix, stg) in csrs:
                nx = do_hits(c, par, list_ref, prefix, stg, nx)
            issue_chunk(c + 2, tt, tl)
            return nx

        return lax.fori_loop(c_lo, c_hi, cbody, nx0)

    prefilter(uidx, listU, cntU, prefU, wrkU)
    prefilter(iidx, listI, cntI, prefI, wrkI)
    prefilter(jidx, listJ, cntJ, prefJ, wrkJ)

    nx = run_pass(tt_u, tl_u, [(listU, prefU, stg_u)], 0)
    nx = run_pass(tt_i, tl_i, [(listI, prefI, stg_i),
                               (listJ, prefJ, stg_j)], nx)

    rem = lax.rem(nx, RING)
    for k in range(RING):
        @pl.when(k < rem)
        def _():
            pltpu.make_async_copy(xfer.at[pl.ds(0, FACTORS)],
                                  stg_j.at[pl.ds(0, FACTORS)], xsem).wait()


def _phase2_body(stg_u, stg_i, stg_j, b1, iidx, jidx, out_hbm,
                 u_buf, i_buf, j_buf, ii_v, jj_v, bi_v, bj_v, out_v, sem):
    b_per_w = ii_v.shape[0]
    wid = lax.axis_index("s") * NUM_CORES + lax.axis_index("c")
    base = wid * b_per_w

    pltpu.sync_copy(stg_u.at[pl.ds(base * FACTORS, b_per_w * FACTORS)], u_buf)
    pltpu.sync_copy(stg_i.at[pl.ds(base * FACTORS, b_per_w * FACTORS)], i_buf)
    pltpu.sync_copy(stg_j.at[pl.ds(base * FACTORS, b_per_w * FACTORS)], j_buf)
    pltpu.sync_copy(iidx.at[pl.ds(base, b_per_w)], ii_v)
    pltpu.sync_copy(jidx.at[pl.ds(base, b_per_w)], jj_v)

    copies = []
    for k in range(b_per_w // 128):
        sl = pl.ds(k * 128, 128)
        copies.append(pltpu.async_copy(b1.at[ii_v.at[sl]], bi_v.at[sl], sem))
        copies.append(pltpu.async_copy(b1.at[jj_v.at[sl]], bj_v.at[sl], sem))
    for c in copies:
        c.wait()

    iota16 = lax.iota(jnp.int32, LANES)

    def group_body(g, _):
        rbase = g * LANES
        acc = bi_v[pl.ds(rbase, LANES)] - bj_v[pl.ds(rbase, LANES)]
        flat0 = (rbase + iota16) * FACTORS
        for f in range(FACTORS):
            flat = flat0 + f
            u = plsc.load_gather(u_buf, [flat])
            iv = plsc.load_gather(i_buf, [flat])
            jv = plsc.load_gather(j_buf, [flat])
            acc = acc + u * (iv - jv)
        out_v[pl.ds(rbase, LANES)] = acc
        return 0

    lax.fori_loop(0, b_per_w // LANES, group_body, 0)

    pltpu.sync_copy(out_v, out_hbm.at[pl.ds(base, b_per_w)])


def _make_phase1():
    mesh = plsc.VectorSubcoreMesh(core_axis_name="c", subcore_axis_name="s")
    return pl.kernel(
        _phase1_body,
        mesh=mesh,
        compiler_params=pltpu.CompilerParams(needs_layout_passes=False),
        out_type=(jax.ShapeDtypeStruct((B * FACTORS,), jnp.float32),
                  jax.ShapeDtypeStruct((B * FACTORS,), jnp.float32),
                  jax.ShapeDtypeStruct((B * FACTORS,), jnp.float32)),
        scratch_types=[
            pltpu.VMEM((B,), jnp.int32),
            pltpu.VMEM((LPAD,), jnp.int32),
            pltpu.VMEM((LPAD,), jnp.int32),
            pltpu.VMEM((LPAD,), jnp.int32),
            pltpu.VMEM((LANES,), jnp.int32),
            pltpu.VMEM((256,), jnp.int32),
            pltpu.VMEM((272,), jnp.int32),
            pltpu.VMEM((256,), jnp.int32),
            pltpu.VMEM((256,), jnp.int32),
            pltpu.VMEM((272,), jnp.int32),
            pltpu.VMEM((256,), jnp.int32),
            pltpu.VMEM((256,), jnp.int32),
            pltpu.VMEM((272,), jnp.int32),
            pltpu.VMEM((256,), jnp.int32),
            pltpu.VMEM((2, FACTORS, CW), jnp.float32),
            pltpu.VMEM((RING * FACTORS,), jnp.float32),
            pltpu.SemaphoreType.DMA,
            pltpu.SemaphoreType.DMA,
            pltpu.SemaphoreType.DMA,
        ],
    )


def _make_phase2():
    b_per_w = B // NUM_WORKERS
    mesh = plsc.VectorSubcoreMesh(core_axis_name="c", subcore_axis_name="s")
    return pl.kernel(
        _phase2_body,
        mesh=mesh,
        compiler_params=pltpu.CompilerParams(needs_layout_passes=False),
        out_type=jax.ShapeDtypeStruct((B,), jnp.float32),
        scratch_types=[
            pltpu.VMEM((b_per_w * FACTORS,), jnp.float32),
            pltpu.VMEM((b_per_w * FACTORS,), jnp.float32),
            pltpu.VMEM((b_per_w * FACTORS,), jnp.float32),
            pltpu.VMEM((b_per_w,), jnp.int32),
            pltpu.VMEM((b_per_w,), jnp.int32),
            pltpu.VMEM((b_per_w,), jnp.float32),
            pltpu.VMEM((b_per_w,), jnp.float32),
            pltpu.VMEM((b_per_w,), jnp.float32),
            pltpu.SemaphoreType.DMA,
        ],
    )


def kernel(user, item_i, item_j, user_table, item_table, item_bias_table):
    u32 = user.astype(jnp.int32)
    i32 = item_i.astype(jnp.int32)
    j32 = item_j.astype(jnp.int32)

    tt_u = user_table.T
    tt_i = item_table.T
    tl_u = jnp.pad(user_table[TAIL_BASE:].T,
                   ((0, 0), (0, CW - (V - TAIL_BASE))))
    tl_i = jnp.pad(item_table[TAIL_BASE:].T,
                   ((0, 0), (0, CW - (V - TAIL_BASE))))
    b1 = item_bias_table.reshape(-1)

    stg_u, stg_i, stg_j = _make_phase1()(
        tt_u, tt_i, tl_u, tl_i, u32, i32, j32)
    return _make_phase2()(stg_u, stg_i, stg_j, b1, i32, j32)


# speedup vs baseline: 5.2356x; 1.2786x over previous
import functools

import jax
import jax.numpy as jnp
from jax import lax
from jax.experimental import pallas as pl
from jax.experimental.pallas import tpu as pltpu
from jax.experimental.pallas import tpu_sc as plsc

NUM_CORES = 2
NUM_SUBCORES = 16
NUM_WORKERS = NUM_CORES * NUM_SUBCORES
LANES = 16
FACTORS = 64
B = 16384
V = 1000000
CW = 128
NCH = V // CW + 1
TAIL_BASE = (V // CW) * CW
CPW = -(-NCH // NUM_WORKERS)
RING = 64
DEPTH = 4
LPAD = B + 32
SENTINEL = jnp.int32(2**30)


def _phase1_body(tt_u, tt_i, tl_u, tl_i, uidx, iidx, jidx,
                 stg_u, stg_i, stg_j,
                 idx_buf, listU, listI, listJ, tmp16,
                 cntU, prefU, wrkU, cntI, prefI, wrkI, cntJ, prefJ, wrkJ,
                 chunk3, xfer, csem0, csem1, csem2, csem3, xsem):
    csems = (csem0, csem1, csem2, csem3)
    wid = lax.axis_index("s") * NUM_CORES + lax.axis_index("c")
    c_lo = wid * CPW
    c_hi = jnp.minimum(c_lo + CPW, NCH)
    base_u = c_lo * CW
    hi_u = c_hi * CW
    iota16 = lax.iota(jnp.int32, LANES)
    mask0 = iota16 == 0

    def popcnt(m):
        return plsc.all_reduce_population_count(m)[0]

    def ffs(m):
        return plsc.all_reduce_ffs(m)[0]

    def prefilter(src_hbm, list_ref, counts, prefix, wrk):
        pltpu.sync_copy(src_hbm, idx_buf)
        zeros16 = jnp.zeros((LANES,), jnp.int32)
        ones16 = jnp.ones((LANES,), jnp.int32)
        for i in range(16):
            counts[pl.ds(i * LANES, LANES)] = zeros16

        def cnt(i, _):
            v = idx_buf[pl.ds(i * LANES, LANES)]
            m = jnp.logical_and(v >= base_u, v < hi_u)
            cidl = lax.shift_right_logical(v - base_u, 7)
            plsc.addupdate_scatter(counts, [cidl], ones16, mask=m)
            return 0

        lax.fori_loop(0, B // LANES, cnt, 0)

        plsc.store_scatter(prefix, [iota16], zeros16, mask=mask0)
        running = jnp.int32(0)
        for i in range(16):
            c16 = counts[pl.ds(i * LANES, LANES)]
            cs = plsc.cumsum(c16) + running
            plsc.store_scatter(prefix, [i * LANES + 1 + iota16], cs)
            running = cs[15]
        for i in range(16):
            wrk[pl.ds(i * LANES, LANES)] = prefix[pl.ds(i * LANES, LANES)]

        def place(i, _):
            v = idx_buf[pl.ds(i * LANES, LANES)]
            m = jnp.logical_and(v >= base_u, v < hi_u)
            packed = jnp.left_shift(v - base_u, 14) | (i * LANES + iota16)
            tmp16[pl.ds(0, LANES)] = packed

            def put(_, mm):
                lane = ffs(mm)
                lv = jnp.full((LANES,), lane, jnp.int32)
                pk = plsc.load_gather(tmp16, [lv])
                cidl = lax.shift_right_logical(pk, 21)
                pos = plsc.load_gather(wrk, [cidl])
                plsc.store_scatter(list_ref, [pos], pk, mask=mask0)
                plsc.store_scatter(wrk, [cidl], pos + 1, mask=mask0)
                return jnp.logical_and(mm, iota16 != lane)

            lax.fori_loop(0, popcnt(m), put, m)
            return 0

        lax.fori_loop(0, B // LANES, place, 0)

    def issue_chunk(c2, tt, tl):
        par = lax.rem(c2 - c_lo, DEPTH)
        for k in range(DEPTH):
            @pl.when(jnp.logical_and(c2 < c_hi, par == k))
            def _(k=k):
                @pl.when(c2 < NCH - 1)
                def _():
                    pltpu.async_copy(tt.at[:, pl.ds(c2 * CW, CW)],
                                     chunk3.at[k], csems[k])

                @pl.when(c2 == NCH - 1)
                def _():
                    pltpu.async_copy(tl.at[:, :], chunk3.at[k], csems[k])

    def wait_chunk(c):
        par = lax.rem(c - c_lo, DEPTH)
        for k in range(DEPTH):
            @pl.when(par == k)
            def _(k=k):
                pltpu.make_async_copy(tl_u.at[:, :], chunk3.at[k],
                                      csems[k]).wait()

    def do_hits(c, par, list_ref, prefix, stg, nx0):
        cv = jnp.full((LANES,), c - c_lo, jnp.int32)
        s0 = plsc.load_gather(prefix, [cv])[0]
        s1 = plsc.load_gather(prefix, [cv + 1])[0]
        parv = jnp.full((LANES,), par, jnp.int32)

        def hh(q, nx2):
            pj = plsc.load_gather(
                list_ref, [jnp.full((LANES,), q, jnp.int32)])[0]
            col = (pj >> 14) - (c - c_lo) * CW
            bpos = pj & (2**14 - 1)
            col_v = jnp.full((LANES,), col, jnp.int32)
            slot = lax.rem(nx2, RING)
            for t in range(4):
                xfer[pl.ds(slot * FACTORS + t * LANES, LANES)] = (
                    plsc.load_gather(
                        chunk3, [parv, t * LANES + iota16, col_v]))
            pltpu.async_copy(
                xfer.at[pl.ds(slot * FACTORS, FACTORS)],
                stg.at[pl.ds(bpos * FACTORS, FACTORS)], xsem)

            @pl.when(slot == RING - 1)
            def _():
                for _k in range(RING):
                    pltpu.make_async_copy(
                        xfer.at[pl.ds(0, FACTORS)],
                        stg.at[pl.ds(0, FACTORS)], xsem).wait()

            return nx2 + 1

        return lax.fori_loop(s0, s1, hh, nx0)

    def run_pass(tt, tl, csrs, nx0, prologue_done=False):
        if not prologue_done:
            for k in range(DEPTH):
                issue_chunk(c_lo + k, tt, tl)

        def cbody(c, nx):
            par = lax.rem(c - c_lo, DEPTH)
            wait_chunk(c)
            for (list_ref, prefix, stg) in csrs:
                nx = do_hits(c, par, list_ref, prefix, stg, nx)
            issue_chunk(c + DEPTH, tt, tl)
            return nx

        return lax.fori_loop(c_lo, c_hi, cbody, nx0)

    for k in range(DEPTH):
        issue_chunk(c_lo + k, tt_u, tl_u)
    prefilter(uidx, listU, cntU, prefU, wrkU)
    prefilter(iidx, listI, cntI, prefI, wrkI)
    prefilter(jidx, listJ, cntJ, prefJ, wrkJ)

    nx = run_pass(tt_u, tl_u, [(listU, prefU, stg_u)], 0,
                  prologue_done=True)
    nx = run_pass(tt_i, tl_i, [(listI, prefI, stg_i),
                               (listJ, prefJ, stg_j)], nx)

    rem = lax.rem(nx, RING)
    for k in range(RING):
        @pl.when(k < rem)
        def _():
            pltpu.make_async_copy(xfer.at[pl.ds(0, FACTORS)],
                                  stg_j.at[pl.ds(0, FACTORS)], xsem).wait()


def _phase2_body(stg_u, stg_i, stg_j, b1, iidx, jidx, out_hbm,
                 u_buf, i_buf, j_buf, ii_v, jj_v, bi_v, bj_v, out_v, sem):
    b_per_w = ii_v.shape[0]
    wid = lax.axis_index("s") * NUM_CORES + lax.axis_index("c")
    base = wid * b_per_w

    pltpu.sync_copy(stg_u.at[pl.ds(base * FACTORS, b_per_w * FACTORS)], u_buf)
    pltpu.sync_copy(stg_i.at[pl.ds(base * FACTORS, b_per_w * FACTORS)], i_buf)
    pltpu.sync_copy(stg_j.at[pl.ds(base * FACTORS, b_per_w * FACTORS)], j_buf)
    pltpu.sync_copy(iidx.at[pl.ds(base, b_per_w)], ii_v)
    pltpu.sync_copy(jidx.at[pl.ds(base, b_per_w)], jj_v)

    copies = []
    for k in range(b_per_w // 128):
        sl = pl.ds(k * 128, 128)
        copies.append(pltpu.async_copy(b1.at[ii_v.at[sl]], bi_v.at[sl], sem))
        copies.append(pltpu.async_copy(b1.at[jj_v.at[sl]], bj_v.at[sl], sem))
    for c in copies:
        c.wait()

    iota16 = lax.iota(jnp.int32, LANES)

    def group_body(g, _):
        rbase = g * LANES
        acc = bi_v[pl.ds(rbase, LANES)] - bj_v[pl.ds(rbase, LANES)]
        flat0 = (rbase + iota16) * FACTORS
        for f in range(FACTORS):
            flat = flat0 + f
            u = plsc.load_gather(u_buf, [flat])
            iv = plsc.load_gather(i_buf, [flat])
            jv = plsc.load_gather(j_buf, [flat])
            acc = acc + u * (iv - jv)
        out_v[pl.ds(rbase, LANES)] = acc
        return 0

    lax.fori_loop(0, b_per_w // LANES, group_body, 0)

    pltpu.sync_copy(out_v, out_hbm.at[pl.ds(base, b_per_w)])


def _make_phase1():
    mesh = plsc.VectorSubcoreMesh(core_axis_name="c", subcore_axis_name="s")
    return pl.kernel(
        _phase1_body,
        mesh=mesh,
        compiler_params=pltpu.CompilerParams(needs_layout_passes=False),
        out_type=(jax.ShapeDtypeStruct((B * FACTORS,), jnp.float32),
                  jax.ShapeDtypeStruct((B * FACTORS,), jnp.float32),
                  jax.ShapeDtypeStruct((B * FACTORS,), jnp.float32)),
        scratch_types=[
            pltpu.VMEM((B,), jnp.int32),
            pltpu.VMEM((LPAD,), jnp.int32),
            pltpu.VMEM((LPAD,), jnp.int32),
            pltpu.VMEM((LPAD,), jnp.int32),
            pltpu.VMEM((LANES,), jnp.int32),
            pltpu.VMEM((256,), jnp.int32),
            pltpu.VMEM((272,), jnp.int32),
            pltpu.VMEM((256,), jnp.int32),
            pltpu.VMEM((256,), jnp.int32),
            pltpu.VMEM((272,), jnp.int32),
            pltpu.VMEM((256,), jnp.int32),
            pltpu.VMEM((256,), jnp.int32),
            pltpu.VMEM((272,), jnp.int32),
            pltpu.VMEM((256,), jnp.int32),
            pltpu.VMEM((DEPTH, FACTORS, CW), jnp.float32),
            pltpu.VMEM((RING * FACTORS,), jnp.float32),
            pltpu.SemaphoreType.DMA,
            pltpu.SemaphoreType.DMA,
            pltpu.SemaphoreType.DMA,
            pltpu.SemaphoreType.DMA,
            pltpu.SemaphoreType.DMA,
        ],
    )


def _make_phase2():
    b_per_w = B // NUM_WORKERS
    mesh = plsc.VectorSubcoreMesh(core_axis_name="c", subcore_axis_name="s")
    return pl.kernel(
        _phase2_body,
        mesh=mesh,
        compiler_params=pltpu.CompilerParams(needs_layout_passes=False),
        out_type=jax.ShapeDtypeStruct((B,), jnp.float32),
        scratch_types=[
            pltpu.VMEM((b_per_w * FACTORS,), jnp.float32),
            pltpu.VMEM((b_per_w * FACTORS,), jnp.float32),
            pltpu.VMEM((b_per_w * FACTORS,), jnp.float32),
            pltpu.VMEM((b_per_w,), jnp.int32),
            pltpu.VMEM((b_per_w,), jnp.int32),
            pltpu.VMEM((b_per_w,), jnp.float32),
            pltpu.VMEM((b_per_w,), jnp.float32),
            pltpu.VMEM((b_per_w,), jnp.float32),
            pltpu.SemaphoreType.DMA,
        ],
    )


def kernel(user, item_i, item_j, user_table, item_table, item_bias_table):
    u32 = user.astype(jnp.int32)
    i32 = item_i.astype(jnp.int32)
    j32 = item_j.astype(jnp.int32)

    tt_u = user_table.T
    tt_i = item_table.T
    tl_u = jnp.pad(user_table[TAIL_BASE:].T,
                   ((0, 0), (0, CW - (V - TAIL_BASE))))
    tl_i = jnp.pad(item_table[TAIL_BASE:].T,
                   ((0, 0), (0, CW - (V - TAIL_BASE))))
    b1 = item_bias_table.reshape(-1)

    stg_u, stg_i, stg_j = _make_phase1()(
        tt_u, tt_i, tl_u, tl_i, u32, i32, j32)
    return _make_phase2()(stg_u, stg_i, stg_j, b1, i32, j32)


# speedup vs baseline: 5.2606x; 1.0048x over previous
import functools

import jax
import jax.numpy as jnp
from jax import lax
from jax.experimental import pallas as pl
from jax.experimental.pallas import tpu as pltpu
from jax.experimental.pallas import tpu_sc as plsc

NUM_CORES = 2
NUM_SUBCORES = 16
NUM_WORKERS = NUM_CORES * NUM_SUBCORES
LANES = 16
FACTORS = 64
B = 16384
V = 1000000
CW = 128
NCH = V // CW + 1
TAIL_BASE = (V // CW) * CW
CPW = -(-NCH // NUM_WORKERS)
RING = 64
DEPTH = 4
LPAD = B + 32
SENTINEL = jnp.int32(2**30)


def _phase1_body(tt_u, tt_i, tl_u, tl_i, uidx, iidx, jidx,
                 stg_u, stg_i, stg_j,
                 idx_buf, listU, listI, listJ, tmp16,
                 cntU, prefU, wrkU, cntI, prefI, wrkI, cntJ, prefJ, wrkJ,
                 chunk3, xfer, csem0, csem1, csem2, csem3, xsem):
    csems = (csem0, csem1, csem2, csem3)
    wid = lax.axis_index("s") * NUM_CORES + lax.axis_index("c")
    c_lo = wid * CPW
    c_hi = jnp.minimum(c_lo + CPW, NCH)
    base_u = c_lo * CW
    hi_u = c_hi * CW
    iota16 = lax.iota(jnp.int32, LANES)
    mask0 = iota16 == 0

    def popcnt(m):
        return plsc.all_reduce_population_count(m)[0]

    def ffs(m):
        return plsc.all_reduce_ffs(m)[0]

    def prefilter(src_hbm, list_ref, counts, prefix, wrk):
        pltpu.sync_copy(src_hbm, idx_buf)
        zeros16 = jnp.zeros((LANES,), jnp.int32)
        ones16 = jnp.ones((LANES,), jnp.int32)
        for i in range(16):
            counts[pl.ds(i * LANES, LANES)] = zeros16

        def cnt(i, _):
            v = idx_buf[pl.ds(i * LANES, LANES)]
            m = jnp.logical_and(v >= base_u, v < hi_u)
            cidl = lax.shift_right_logical(v - base_u, 7)
            plsc.addupdate_scatter(counts, [cidl], ones16, mask=m)
            return 0

        lax.fori_loop(0, B // LANES, cnt, 0)

        plsc.store_scatter(prefix, [iota16], zeros16, mask=mask0)
        running = jnp.int32(0)
        for i in range(16):
            c16 = counts[pl.ds(i * LANES, LANES)]
            cs = plsc.cumsum(c16) + running
            plsc.store_scatter(prefix, [i * LANES + 1 + iota16], cs)
            running = cs[15]
        for i in range(16):
            wrk[pl.ds(i * LANES, LANES)] = prefix[pl.ds(i * LANES, LANES)]

        def place(i, _):
            v = idx_buf[pl.ds(i * LANES, LANES)]
            m = jnp.logical_and(v >= base_u, v < hi_u)
            packed = jnp.left_shift(v - base_u, 14) | (i * LANES + iota16)
            tmp16[pl.ds(0, LANES)] = packed

            def put(_, mm):
                lane = ffs(mm)
                lv = jnp.full((LANES,), lane, jnp.int32)
                pk = plsc.load_gather(tmp16, [lv])
                cidl = lax.shift_right_logical(pk, 21)
                pos = plsc.load_gather(wrk, [cidl])
                plsc.store_scatter(list_ref, [pos], pk, mask=mask0)
                plsc.store_scatter(wrk, [cidl], pos + 1, mask=mask0)
                return jnp.logical_and(mm, iota16 != lane)

            lax.fori_loop(0, popcnt(m), put, m)
            return 0

        lax.fori_loop(0, B // LANES, place, 0)

    def chunk_live(c, prefixes):
        n = jnp.int32(0)
        cv0 = jnp.full((LANES,), c - c_lo, jnp.int32)
        for prefix in prefixes:
            n = n + (plsc.load_gather(prefix, [cv0 + 1])[0]
                     - plsc.load_gather(prefix, [cv0])[0])
        return n > 0

    def issue_chunk(c2, tt, tl, prefixes):
        par = lax.rem(c2 - c_lo, DEPTH)
        live = jnp.logical_and(c2 < c_hi, chunk_live(c2, prefixes))
        for k in range(DEPTH):
            @pl.when(jnp.logical_and(live, par == k))
            def _(k=k):
                @pl.when(c2 < NCH - 1)
                def _():
                    pltpu.async_copy(tt.at[:, pl.ds(c2 * CW, CW)],
                                     chunk3.at[k], csems[k])

                @pl.when(c2 == NCH - 1)
                def _():
                    pltpu.async_copy(tl.at[:, :], chunk3.at[k], csems[k])

    def wait_chunk(c, prefixes):
        par = lax.rem(c - c_lo, DEPTH)
        for k in range(DEPTH):
            @pl.when(jnp.logical_and(chunk_live(c, prefixes), par == k))
            def _(k=k):
                pltpu.make_async_copy(tl_u.at[:, :], chunk3.at[k],
                                      csems[k]).wait()

    def do_hits(c, par, list_ref, prefix, stg, nx0):
        cv = jnp.full((LANES,), c - c_lo, jnp.int32)
        s0 = plsc.load_gather(prefix, [cv])[0]
        s1 = plsc.load_gather(prefix, [cv + 1])[0]
        parv = jnp.full((LANES,), par, jnp.int32)

        def hh(q, nx2):
            pj = plsc.load_gather(
                list_ref, [jnp.full((LANES,), q, jnp.int32)])[0]
            col = (pj >> 14) - (c - c_lo) * CW
            bpos = pj & (2**14 - 1)
            col_v = jnp.full((LANES,), col, jnp.int32)
            slot = lax.rem(nx2, RING)
            for t in range(4):
                xfer[pl.ds(slot * FACTORS + t * LANES, LANES)] = (
                    plsc.load_gather(
                        chunk3, [parv, t * LANES + iota16, col_v]))
            pltpu.async_copy(
                xfer.at[pl.ds(slot * FACTORS, FACTORS)],
                stg.at[pl.ds(bpos * FACTORS, FACTORS)], xsem)

            @pl.when(slot == RING - 1)
            def _():
                for _k in range(RING):
                    pltpu.make_async_copy(
                        xfer.at[pl.ds(0, FACTORS)],
                        stg.at[pl.ds(0, FACTORS)], xsem).wait()

            return nx2 + 1

        return lax.fori_loop(s0, s1, hh, nx0)

    def run_pass(tt, tl, csrs, nx0, prologue_done=False):
        prefixes = [prefix for (_, prefix, _) in csrs]
        if not prologue_done:
            for k in range(DEPTH):
                issue_chunk(c_lo + k, tt, tl, prefixes)

        def cbody(c, nx):
            par = lax.rem(c - c_lo, DEPTH)
            wait_chunk(c, prefixes)
            for (list_ref, prefix, stg) in csrs:
                nx = do_hits(c, par, list_ref, prefix, stg, nx)
            issue_chunk(c + DEPTH, tt, tl, prefixes)
            return nx

        return lax.fori_loop(c_lo, c_hi, cbody, nx0)

    prefilter(uidx, listU, cntU, prefU, wrkU)
    for k in range(DEPTH):
        issue_chunk(c_lo + k, tt_u, tl_u, [prefU])
    prefilter(iidx, listI, cntI, prefI, wrkI)
    prefilter(jidx, listJ, cntJ, prefJ, wrkJ)

    nx = run_pass(tt_u, tl_u, [(listU, prefU, stg_u)], 0,
                  prologue_done=True)
    nx = run_pass(tt_i, tl_i, [(listI, prefI, stg_i),
                               (listJ, prefJ, stg_j)], nx)

    rem = lax.rem(nx, RING)
    for k in range(RING):
        @pl.when(k < rem)
        def _():
            pltpu.make_async_copy(xfer.at[pl.ds(0, FACTORS)],
                                  stg_j.at[pl.ds(0, FACTORS)], xsem).wait()


def _phase2_body(stg_u, stg_i, stg_j, b1, iidx, jidx, out_hbm,
                 u_buf, i_buf, j_buf, ii_v, jj_v, bi_v, bj_v, out_v, sem):
    b_per_w = ii_v.shape[0]
    wid = lax.axis_index("s") * NUM_CORES + lax.axis_index("c")
    base = wid * b_per_w

    pltpu.sync_copy(stg_u.at[pl.ds(base * FACTORS, b_per_w * FACTORS)], u_buf)
    pltpu.sync_copy(stg_i.at[pl.ds(base * FACTORS, b_per_w * FACTORS)], i_buf)
    pltpu.sync_copy(stg_j.at[pl.ds(base * FACTORS, b_per_w * FACTORS)], j_buf)
    pltpu.sync_copy(iidx.at[pl.ds(base, b_per_w)], ii_v)
    pltpu.sync_copy(jidx.at[pl.ds(base, b_per_w)], jj_v)

    copies = []
    for k in range(b_per_w // 128):
        sl = pl.ds(k * 128, 128)
        copies.append(pltpu.async_copy(b1.at[ii_v.at[sl]], bi_v.at[sl], sem))
        copies.append(pltpu.async_copy(b1.at[jj_v.at[sl]], bj_v.at[sl], sem))
    for c in copies:
        c.wait()

    iota16 = lax.iota(jnp.int32, LANES)

    def group_body(g, _):
        rbase = g * LANES
        acc = bi_v[pl.ds(rbase, LANES)] - bj_v[pl.ds(rbase, LANES)]
        flat0 = (rbase + iota16) * FACTORS
        for f in range(FACTORS):
            flat = flat0 + f
            u = plsc.load_gather(u_buf, [flat])
            iv = plsc.load_gather(i_buf, [flat])
            jv = plsc.load_gather(j_buf, [flat])
            acc = acc + u * (iv - jv)
        out_v[pl.ds(rbase, LANES)] = acc
        return 0

    lax.fori_loop(0, b_per_w // LANES, group_body, 0)

    pltpu.sync_copy(out_v, out_hbm.at[pl.ds(base, b_per_w)])


def _make_phase1():
    mesh = plsc.VectorSubcoreMesh(core_axis_name="c", subcore_axis_name="s")
    return pl.kernel(
        _phase1_body,
        mesh=mesh,
        compiler_params=pltpu.CompilerParams(needs_layout_passes=False),
        out_type=(jax.ShapeDtypeStruct((B * FACTORS,), jnp.float32),
                  jax.ShapeDtypeStruct((B * FACTORS,), jnp.float32),
                  jax.ShapeDtypeStruct((B * FACTORS,), jnp.float32)),
        scratch_types=[
            pltpu.VMEM((B,), jnp.int32),
            pltpu.VMEM((LPAD,), jnp.int32),
            pltpu.VMEM((LPAD,), jnp.int32),
            pltpu.VMEM((LPAD,), jnp.int32),
            pltpu.VMEM((LANES,), jnp.int32),
            pltpu.VMEM((256,), jnp.int32),
            pltpu.VMEM((272,), jnp.int32),
            pltpu.VMEM((256,), jnp.int32),
            pltpu.VMEM((256,), jnp.int32),
            pltpu.VMEM((272,), jnp.int32),
            pltpu.VMEM((256,), jnp.int32),
            pltpu.VMEM((256,), jnp.int32),
            pltpu.VMEM((272,), jnp.int32),
            pltpu.VMEM((256,), jnp.int32),
            pltpu.VMEM((DEPTH, FACTORS, CW), jnp.float32),
            pltpu.VMEM((RING * FACTORS,), jnp.float32),
            pltpu.SemaphoreType.DMA,
            pltpu.SemaphoreType.DMA,
            pltpu.SemaphoreType.DMA,
            pltpu.SemaphoreType.DMA,
            pltpu.SemaphoreType.DMA,
        ],
    )


def _make_phase2():
    b_per_w = B // NUM_WORKERS
    mesh = plsc.VectorSubcoreMesh(core_axis_name="c", subcore_axis_name="s")
    return pl.kernel(
        _phase2_body,
        mesh=mesh,
        compiler_params=pltpu.CompilerParams(needs_layout_passes=False),
        out_type=jax.ShapeDtypeStruct((B,), jnp.float32),
        scratch_types=[
            pltpu.VMEM((b_per_w * FACTORS,), jnp.float32),
            pltpu.VMEM((b_per_w * FACTORS,), jnp.float32),
            pltpu.VMEM((b_per_w * FACTORS,), jnp.float32),
            pltpu.VMEM((b_per_w,), jnp.int32),
            pltpu.VMEM((b_per_w,), jnp.int32),
            pltpu.VMEM((b_per_w,), jnp.float32),
            pltpu.VMEM((b_per_w,), jnp.float32),
            pltpu.VMEM((b_per_w,), jnp.float32),
            pltpu.SemaphoreType.DMA,
        ],
    )


def kernel(user, item_i, item_j, user_table, item_table, item_bias_table):
    u32 = user.astype(jnp.int32)
    i32 = item_i.astype(jnp.int32)
    j32 = item_j.astype(jnp.int32)

    tt_u = user_table.T
    tt_i = item_table.T
    tl_u = jnp.pad(user_table[TAIL_BASE:].T,
                   ((0, 0), (0, CW - (V - TAIL_BASE))))
    tl_i = jnp.pad(item_table[TAIL_BASE:].T,
                   ((0, 0), (0, CW - (V - TAIL_BASE))))
    b1 = item_bias_table.reshape(-1)

    stg_u, stg_i, stg_j = _make_phase1()(
        tt_u, tt_i, tl_u, tl_i, u32, i32, j32)
    return _make_phase2()(stg_u, stg_i, stg_j, b1, i32, j32)


# speedup vs baseline: 5.4774x; 1.0412x over previous
import functools

import jax
import jax.numpy as jnp
from jax import lax
from jax.experimental import pallas as pl
from jax.experimental.pallas import tpu as pltpu
from jax.experimental.pallas import tpu_sc as plsc

NUM_CORES = 2
NUM_SUBCORES = 16
NUM_WORKERS = NUM_CORES * NUM_SUBCORES
LANES = 16
FACTORS = 64
B = 16384
V = 1000000
CW = 128
NCH = V // CW + 1
TAIL_BASE = (V // CW) * CW
CPW = -(-NCH // NUM_WORKERS)
RING = 64
DEPTH = 6
LPAD = B + 32
SENTINEL = jnp.int32(2**30)


def _phase1_body(tt_u, tt_i, tl_u, tl_i, uidx, iidx, jidx,
                 stg_u, stg_i, stg_j,
                 idx_buf, listU, listI, listJ, tmp16,
                 cntU, prefU, wrkU, cntI, prefI, wrkI, cntJ, prefJ, wrkJ,
                 chunk3, xfer, csem0, csem1, csem2, csem3, csem4, csem5,
                 xsem):
    csems = (csem0, csem1, csem2, csem3, csem4, csem5)
    wid = lax.axis_index("s") * NUM_CORES + lax.axis_index("c")
    c_lo = wid * CPW
    c_hi = jnp.minimum(c_lo + CPW, NCH)
    base_u = c_lo * CW
    hi_u = c_hi * CW
    iota16 = lax.iota(jnp.int32, LANES)
    mask0 = iota16 == 0

    def popcnt(m):
        return plsc.all_reduce_population_count(m)[0]

    def ffs(m):
        return plsc.all_reduce_ffs(m)[0]

    def prefilter(src_hbm, list_ref, counts, prefix, wrk):
        pltpu.sync_copy(src_hbm, idx_buf)
        zeros16 = jnp.zeros((LANES,), jnp.int32)
        ones16 = jnp.ones((LANES,), jnp.int32)
        for i in range(16):
            counts[pl.ds(i * LANES, LANES)] = zeros16

        def cnt(i, _):
            v = idx_buf[pl.ds(i * LANES, LANES)]
            m = jnp.logical_and(v >= base_u, v < hi_u)
            cidl = lax.shift_right_logical(v - base_u, 7)
            plsc.addupdate_scatter(counts, [cidl], ones16, mask=m)
            return 0

        lax.fori_loop(0, B // LANES, cnt, 0)

        plsc.store_scatter(prefix, [iota16], zeros16, mask=mask0)
        running = jnp.int32(0)
        for i in range(16):
            c16 = counts[pl.ds(i * LANES, LANES)]
            cs = plsc.cumsum(c16) + running
            plsc.store_scatter(prefix, [i * LANES + 1 + iota16], cs)
            running = cs[15]
        for i in range(16):
            wrk[pl.ds(i * LANES, LANES)] = prefix[pl.ds(i * LANES, LANES)]

        def place(i, _):
            v = idx_buf[pl.ds(i * LANES, LANES)]
            m = jnp.logical_and(v >= base_u, v < hi_u)
            packed = jnp.left_shift(v - base_u, 14) | (i * LANES + iota16)
            tmp16[pl.ds(0, LANES)] = packed

            def put(_, mm):
                lane = ffs(mm)
                lv = jnp.full((LANES,), lane, jnp.int32)
                pk = plsc.load_gather(tmp16, [lv])
                cidl = lax.shift_right_logical(pk, 21)
                pos = plsc.load_gather(wrk, [cidl])
                plsc.store_scatter(list_ref, [pos], pk, mask=mask0)
                plsc.store_scatter(wrk, [cidl], pos + 1, mask=mask0)
                return jnp.logical_and(mm, iota16 != lane)

            lax.fori_loop(0, popcnt(m), put, m)
            return 0

        lax.fori_loop(0, B // LANES, place, 0)

    def chunk_live(c, prefixes):
        n = jnp.int32(0)
        cv0 = jnp.full((LANES,), c - c_lo, jnp.int32)
        for prefix in prefixes:
            n = n + (plsc.load_gather(prefix, [cv0 + 1])[0]
                     - plsc.load_gather(prefix, [cv0])[0])
        return n > 0

    def issue_chunk(c2, tt, tl, prefixes):
        par = lax.rem(c2 - c_lo, DEPTH)
        live = jnp.logical_and(c2 < c_hi, chunk_live(c2, prefixes))
        for k in range(DEPTH):
            @pl.when(jnp.logical_and(live, par == k))
            def _(k=k):
                @pl.when(c2 < NCH - 1)
                def _():
                    pltpu.async_copy(tt.at[:, pl.ds(c2 * CW, CW)],
                                     chunk3.at[k], csems[k])

                @pl.when(c2 == NCH - 1)
                def _():
                    pltpu.async_copy(tl.at[:, :], chunk3.at[k], csems[k])

    def wait_chunk(c, prefixes):
        par = lax.rem(c - c_lo, DEPTH)
        for k in range(DEPTH):
            @pl.when(jnp.logical_and(chunk_live(c, prefixes), par == k))
            def _(k=k):
                pltpu.make_async_copy(tl_u.at[:, :], chunk3.at[k],
                                      csems[k]).wait()

    def do_hits(c, par, list_ref, prefix, stg, nx0):
        cv = jnp.full((LANES,), c - c_lo, jnp.int32)
        s0 = plsc.load_gather(prefix, [cv])[0]
        s1 = plsc.load_gather(prefix, [cv + 1])[0]
        parv = jnp.full((LANES,), par, jnp.int32)

        def hh(q, nx2):
            pj = plsc.load_gather(
                list_ref, [jnp.full((LANES,), q, jnp.int32)])[0]
            col = (pj >> 14) - (c - c_lo) * CW
            bpos = pj & (2**14 - 1)
            col_v = jnp.full((LANES,), col, jnp.int32)
            slot = lax.rem(nx2, RING)
            for t in range(4):
                xfer[pl.ds(slot * FACTORS + t * LANES, LANES)] = (
                    plsc.load_gather(
                        chunk3, [parv, t * LANES + iota16, col_v]))
            pltpu.async_copy(
                xfer.at[pl.ds(slot * FACTORS, FACTORS)],
                stg.at[pl.ds(bpos * FACTORS, FACTORS)], xsem)

            @pl.when(slot == RING - 1)
            def _():
                for _k in range(RING):
                    pltpu.make_async_copy(
                        xfer.at[pl.ds(0, FACTORS)],
                        stg.at[pl.ds(0, FACTORS)], xsem).wait()

            return nx2 + 1

        return lax.fori_loop(s0, s1, hh, nx0)

    def run_pass(tt, tl, csrs, nx0, prologue_done=False):
        prefixes = [prefix for (_, prefix, _) in csrs]
        if not prologue_done:
            for k in range(DEPTH):
                issue_chunk(c_lo + k, tt, tl, prefixes)

        def cbody(c, nx):
            par = lax.rem(c - c_lo, DEPTH)
            wait_chunk(c, prefixes)
            for (list_ref, prefix, stg) in csrs:
                nx = do_hits(c, par, list_ref, prefix, stg, nx)
            issue_chunk(c + DEPTH, tt, tl, prefixes)
            return nx

        return lax.fori_loop(c_lo, c_hi, cbody, nx0)

    prefilter(uidx, listU, cntU, prefU, wrkU)
    for k in range(DEPTH):
        issue_chunk(c_lo + k, tt_u, tl_u, [prefU])
    prefilter(iidx, listI, cntI, prefI, wrkI)
    prefilter(jidx, listJ, cntJ, prefJ, wrkJ)

    nx = run_pass(tt_u, tl_u, [(listU, prefU, stg_u)], 0,
                  prologue_done=True)
    nx = run_pass(tt_i, tl_i, [(listI, prefI, stg_i),
                               (listJ, prefJ, stg_j)], nx)

    rem = lax.rem(nx, RING)
    for k in range(RING):
        @pl.when(k < rem)
        def _():
            pltpu.make_async_copy(xfer.at[pl.ds(0, FACTORS)],
                                  stg_j.at[pl.ds(0, FACTORS)], xsem).wait()


def _phase2_body(stg_u, stg_i, stg_j, b1, iidx, jidx, out_hbm,
                 u_buf, i_buf, j_buf, ii_v, jj_v, bi_v, bj_v, out_v, sem):
    b_per_w = ii_v.shape[0]
    wid = lax.axis_index("s") * NUM_CORES + lax.axis_index("c")
    base = wid * b_per_w

    pltpu.sync_copy(stg_u.at[pl.ds(base * FACTORS, b_per_w * FACTORS)], u_buf)
    pltpu.sync_copy(stg_i.at[pl.ds(base * FACTORS, b_per_w * FACTORS)], i_buf)
    pltpu.sync_copy(stg_j.at[pl.ds(base * FACTORS, b_per_w * FACTORS)], j_buf)
    pltpu.sync_copy(iidx.at[pl.ds(base, b_per_w)], ii_v)
    pltpu.sync_copy(jidx.at[pl.ds(base, b_per_w)], jj_v)

    copies = []
    for k in range(b_per_w // 128):
        sl = pl.ds(k * 128, 128)
        copies.append(pltpu.async_copy(b1.at[ii_v.at[sl]], bi_v.at[sl], sem))
        copies.append(pltpu.async_copy(b1.at[jj_v.at[sl]], bj_v.at[sl], sem))
    for c in copies:
        c.wait()

    iota16 = lax.iota(jnp.int32, LANES)

    def group_body(g, _):
        rbase = g * LANES
        acc = bi_v[pl.ds(rbase, LANES)] - bj_v[pl.ds(rbase, LANES)]
        flat0 = (rbase + iota16) * FACTORS
        for f in range(FACTORS):
            flat = flat0 + f
            u = plsc.load_gather(u_buf, [flat])
            iv = plsc.load_gather(i_buf, [flat])
            jv = plsc.load_gather(j_buf, [flat])
            acc = acc + u * (iv - jv)
        out_v[pl.ds(rbase, LANES)] = acc
        return 0

    lax.fori_loop(0, b_per_w // LANES, group_body, 0)

    pltpu.sync_copy(out_v, out_hbm.at[pl.ds(base, b_per_w)])


def _make_phase1():
    mesh = plsc.VectorSubcoreMesh(core_axis_name="c", subcore_axis_name="s")
    return pl.kernel(
        _phase1_body,
        mesh=mesh,
        compiler_params=pltpu.CompilerParams(needs_layout_passes=False),
        out_type=(jax.ShapeDtypeStruct((B * FACTORS,), jnp.float32),
                  jax.ShapeDtypeStruct((B * FACTORS,), jnp.float32),
                  jax.ShapeDtypeStruct((B * FACTORS,), jnp.float32)),
        scratch_types=[
            pltpu.VMEM((B,), jnp.int32),
            pltpu.VMEM((LPAD,), jnp.int32),
            pltpu.VMEM((LPAD,), jnp.int32),
            pltpu.VMEM((LPAD,), jnp.int32),
            pltpu.VMEM((LANES,), jnp.int32),
            pltpu.VMEM((256,), jnp.int32),
            pltpu.VMEM((272,), jnp.int32),
            pltpu.VMEM((256,), jnp.int32),
            pltpu.VMEM((256,), jnp.int32),
            pltpu.VMEM((272,), jnp.int32),
            pltpu.VMEM((256,), jnp.int32),
            pltpu.VMEM((256,), jnp.int32),
            pltpu.VMEM((272,), jnp.int32),
            pltpu.VMEM((256,), jnp.int32),
            pltpu.VMEM((DEPTH, FACTORS, CW), jnp.float32),
            pltpu.VMEM((RING * FACTORS,), jnp.float32),
            pltpu.SemaphoreType.DMA,
            pltpu.SemaphoreType.DMA,
            pltpu.SemaphoreType.DMA,
            pltpu.SemaphoreType.DMA,
            pltpu.SemaphoreType.DMA,
            pltpu.SemaphoreType.DMA,
            pltpu.SemaphoreType.DMA,
        ],
    )


def _make_phase2():
    b_per_w = B // NUM_WORKERS
    mesh = plsc.VectorSubcoreMesh(core_axis_name="c", subcore_axis_name="s")
    return pl.kernel(
        _phase2_body,
        mesh=mesh,
        compiler_params=pltpu.CompilerParams(needs_layout_passes=False),
        out_type=jax.ShapeDtypeStruct((B,), jnp.float32),
        scratch_types=[
            pltpu.VMEM((b_per_w * FACTORS,), jnp.float32),
            pltpu.VMEM((b_per_w * FACTORS,), jnp.float32),
            pltpu.VMEM((b_per_w * FACTORS,), jnp.float32),
            pltpu.VMEM((b_per_w,), jnp.int32),
            pltpu.VMEM((b_per_w,), jnp.int32),
            pltpu.VMEM((b_per_w,), jnp.float32),
            pltpu.VMEM((b_per_w,), jnp.float32),
            pltpu.VMEM((b_per_w,), jnp.float32),
            pltpu.SemaphoreType.DMA,
        ],
    )


def kernel(user, item_i, item_j, user_table, item_table, item_bias_table):
    u32 = user.astype(jnp.int32)
    i32 = item_i.astype(jnp.int32)
    j32 = item_j.astype(jnp.int32)

    tt_u = user_table.T
    tt_i = item_table.T
    tl_u = jnp.pad(user_table[TAIL_BASE:].T,
                   ((0, 0), (0, CW - (V - TAIL_BASE))))
    tl_i = jnp.pad(item_table[TAIL_BASE:].T,
                   ((0, 0), (0, CW - (V - TAIL_BASE))))
    b1 = item_bias_table.reshape(-1)

    stg_u, stg_i, stg_j = _make_phase1()(
        tt_u, tt_i, tl_u, tl_i, u32, i32, j32)
    return _make_phase2()(stg_u, stg_i, stg_j, b1, i32, j32)


# speedup vs baseline: 5.4876x; 1.0018x over previous
import jax
import jax.numpy as jnp
from jax import lax
from jax.experimental import pallas as pl
from jax.experimental.pallas import tpu as pltpu
from jax.experimental.pallas import tpu_sc as plsc

NUM_CORES = 2
NUM_SUBCORES = 16
NUM_WORKERS = NUM_CORES * NUM_SUBCORES
LANES = 16
FACTORS = 64
B = 16384
V = 1000000
CW = 128
NCH = V // CW + 1
TAIL_BASE = (V // CW) * CW
CPW = -(-NCH // NUM_WORKERS)
RING = 64
DEPTH = 6
LPAD = B + 32


def _phase1_body(tt_u, tt_i, tl_u, tl_i, uidx, iidx, jidx,
                 stg_u, stg_i, stg_j,
                 idx_buf, listU, listI, listJ, tmp16,
                 cntU, prefU, wrkU, cntI, prefI, wrkI, cntJ, prefJ, wrkJ,
                 chunk3, xfer, csem0, csem1, csem2, csem3, csem4, csem5,
                 xsem):
    csems = (csem0, csem1, csem2, csem3, csem4, csem5)
    wid = lax.axis_index("s") * NUM_CORES + lax.axis_index("c")
    c_lo = wid * CPW
    c_hi = jnp.minimum(c_lo + CPW, NCH)
    base_u = c_lo * CW
    hi_u = c_hi * CW
    iota16 = lax.iota(jnp.int32, LANES)
    mask0 = iota16 == 0

    def popcnt(m):
        return plsc.all_reduce_population_count(m)[0]

    def ffs(m):
        return plsc.all_reduce_ffs(m)[0]

    def prefilter(src_hbm, list_ref, counts, prefix, wrk):
        pltpu.sync_copy(src_hbm, idx_buf)
        zeros16 = jnp.zeros((LANES,), jnp.int32)
        ones16 = jnp.ones((LANES,), jnp.int32)
        for i in range(16):
            counts[pl.ds(i * LANES, LANES)] = zeros16

        def cnt(i, _):
            v = idx_buf[pl.ds(i * LANES, LANES)]
            m = jnp.logical_and(v >= base_u, v < hi_u)
            cidl = lax.shift_right_logical(v - base_u, 7)
            plsc.addupdate_scatter(counts, [cidl], ones16, mask=m)
            return 0

        lax.fori_loop(0, B // LANES, cnt, 0)

        plsc.store_scatter(prefix, [iota16], zeros16, mask=mask0)
        running = jnp.int32(0)
        for i in range(16):
            c16 = counts[pl.ds(i * LANES, LANES)]
            cs = plsc.cumsum(c16) + running
            plsc.store_scatter(prefix, [i * LANES + 1 + iota16], cs)
            running = cs[15]
        for i in range(16):
            wrk[pl.ds(i * LANES, LANES)] = prefix[pl.ds(i * LANES, LANES)]

        def place(i, _):
            v = idx_buf[pl.ds(i * LANES, LANES)]
            m = jnp.logical_and(v >= base_u, v < hi_u)
            packed = jnp.left_shift(v - base_u, 14) | (i * LANES + iota16)
            tmp16[pl.ds(0, LANES)] = packed

            def put(_, mm):
                lane = ffs(mm)
                lv = jnp.full((LANES,), lane, jnp.int32)
                pk = plsc.load_gather(tmp16, [lv])
                cidl = lax.shift_right_logical(pk, 21)
                pos = plsc.load_gather(wrk, [cidl])
                plsc.store_scatter(list_ref, [pos], pk, mask=mask0)
                plsc.store_scatter(wrk, [cidl], pos + 1, mask=mask0)
                return jnp.logical_and(mm, iota16 != lane)

            lax.fori_loop(0, popcnt(m), put, m)
            return 0

        lax.fori_loop(0, B // LANES, place, 0)

    def chunk_live(c, prefixes):
        n = jnp.int32(0)
        cv0 = jnp.full((LANES,), c - c_lo, jnp.int32)
        for prefix in prefixes:
            n = n + (plsc.load_gather(prefix, [cv0 + 1])[0]
                     - plsc.load_gather(prefix, [cv0])[0])
        return n > 0

    def issue_chunk(c2, tt, tl, prefixes):
        par = lax.rem(c2 - c_lo, DEPTH)
        live = jnp.logical_and(c2 < c_hi, chunk_live(c2, prefixes))
        for k in range(DEPTH):
            @pl.when(jnp.logical_and(live, par == k))
            def _(k=k):
                @pl.when(c2 < NCH - 1)
                def _():
                    pltpu.async_copy(tt.at[:, pl.ds(c2 * CW, CW)],
                                     chunk3.at[k], csems[k])

                @pl.when(c2 == NCH - 1)
                def _():
                    pltpu.async_copy(tl.at[:, :], chunk3.at[k], csems[k])

    def wait_chunk(c, prefixes):
        par = lax.rem(c - c_lo, DEPTH)
        for k in range(DEPTH):
            @pl.when(jnp.logical_and(chunk_live(c, prefixes), par == k))
            def _(k=k):
                pltpu.make_async_copy(tl_u.at[:, :], chunk3.at[k],
                                      csems[k]).wait()

    def do_hits(c, par, list_ref, prefix, stg, nx0):
        cv = jnp.full((LANES,), c - c_lo, jnp.int32)
        s0 = plsc.load_gather(prefix, [cv])[0]
        s1 = plsc.load_gather(prefix, [cv + 1])[0]
        parv = jnp.full((LANES,), par, jnp.int32)

        def hh(q, nx2):
            pj = plsc.load_gather(
                list_ref, [jnp.full((LANES,), q, jnp.int32)])[0]
            col = (pj >> 14) - (c - c_lo) * CW
            bpos = pj & (2**14 - 1)
            col_v = jnp.full((LANES,), col, jnp.int32)
            slot = lax.rem(nx2, RING)
            for t in range(4):
                xfer[pl.ds(slot * FACTORS + t * LANES, LANES)] = (
                    plsc.load_gather(
                        chunk3, [parv, t * LANES + iota16, col_v]))
            pltpu.async_copy(
                xfer.at[pl.ds(slot * FACTORS, FACTORS)],
                stg.at[pl.ds(bpos * FACTORS, FACTORS)], xsem)

            @pl.when(slot == RING - 1)
            def _():
                for _k in range(RING):
                    pltpu.make_async_copy(
                        xfer.at[pl.ds(0, FACTORS)],
                        stg.at[pl.ds(0, FACTORS)], xsem).wait()

            return nx2 + 1

        return lax.fori_loop(s0, s1, hh, nx0)

    def run_pass(tt, tl, csrs, nx0, prologue_done=False):
        prefixes = [prefix for (_, prefix, _) in csrs]
        if not prologue_done:
            for k in range(DEPTH):
                issue_chunk(c_lo + k, tt, tl, prefixes)

        def cbody(c, nx):
            par = lax.rem(c - c_lo, DEPTH)
            wait_chunk(c, prefixes)
            for (list_ref, prefix, stg) in csrs:
                nx = do_hits(c, par, list_ref, prefix, stg, nx)
            issue_chunk(c + DEPTH, tt, tl, prefixes)
            return nx

        return lax.fori_loop(c_lo, c_hi, cbody, nx0)

    prefilter(uidx, listU, cntU, prefU, wrkU)
    for k in range(DEPTH):
        issue_chunk(c_lo + k, tt_u, tl_u, [prefU])
    prefilter(iidx, listI, cntI, prefI, wrkI)
    prefilter(jidx, listJ, cntJ, prefJ, wrkJ)

    nx = run_pass(tt_u, tl_u, [(listU, prefU, stg_u)], 0,
                  prologue_done=True)
    nx = run_pass(tt_i, tl_i, [(listI, prefI, stg_i),
                               (listJ, prefJ, stg_j)], nx)

    rem = lax.rem(nx, RING)
    for k in range(RING):
        @pl.when(k < rem)
        def _():
            pltpu.make_async_copy(xfer.at[pl.ds(0, FACTORS)],
                                  stg_j.at[pl.ds(0, FACTORS)], xsem).wait()


def _phase2_body(stg_u, stg_i, stg_j, b1, iidx, jidx, out_hbm,
                 u_buf, i_buf, j_buf, ii_v, jj_v, bi_v, bj_v, out_v, sem):
    b_per_w = ii_v.shape[0]
    wid = lax.axis_index("s") * NUM_CORES + lax.axis_index("c")
    base = wid * b_per_w

    pltpu.sync_copy(stg_u.at[pl.ds(base * FACTORS, b_per_w * FACTORS)], u_buf)
    pltpu.sync_copy(stg_i.at[pl.ds(base * FACTORS, b_per_w * FACTORS)], i_buf)
    pltpu.sync_copy(stg_j.at[pl.ds(base * FACTORS, b_per_w * FACTORS)], j_buf)
    pltpu.sync_copy(iidx.at[pl.ds(base, b_per_w)], ii_v)
    pltpu.sync_copy(jidx.at[pl.ds(base, b_per_w)], jj_v)

    copies = []
    for k in range(b_per_w // 128):
        sl = pl.ds(k * 128, 128)
        copies.append(pltpu.async_copy(b1.at[ii_v.at[sl]], bi_v.at[sl], sem))
        copies.append(pltpu.async_copy(b1.at[jj_v.at[sl]], bj_v.at[sl], sem))
    for c in copies:
        c.wait()

    iota16 = lax.iota(jnp.int32, LANES)

    def group_body(g, _):
        rbase = g * LANES
        acc = bi_v[pl.ds(rbase, LANES)] - bj_v[pl.ds(rbase, LANES)]
        flat0 = (rbase + iota16) * FACTORS
        for f in range(FACTORS):
            flat = flat0 + f
            u = plsc.load_gather(u_buf, [flat])
            iv = plsc.load_gather(i_buf, [flat])
            jv = plsc.load_gather(j_buf, [flat])
            acc = acc + u * (iv - jv)
        out_v[pl.ds(rbase, LANES)] = acc
        return 0

    lax.fori_loop(0, b_per_w // LANES, group_body, 0)

    pltpu.sync_copy(out_v, out_hbm.at[pl.ds(base, b_per_w)])


def _make_phase1():
    mesh = plsc.VectorSubcoreMesh(core_axis_name="c", subcore_axis_name="s")
    return pl.kernel(
        _phase1_body,
        mesh=mesh,
        compiler_params=pltpu.CompilerParams(needs_layout_passes=False),
        out_type=(jax.ShapeDtypeStruct((B * FACTORS,), jnp.float32),
                  jax.ShapeDtypeStruct((B * FACTORS,), jnp.float32),
                  jax.ShapeDtypeStruct((B * FACTORS,), jnp.float32)),
        scratch_types=[
            pltpu.VMEM((B,), jnp.int32),
            pltpu.VMEM((LPAD,), jnp.int32),
            pltpu.VMEM((LPAD,), jnp.int32),
            pltpu.VMEM((LPAD,), jnp.int32),
            pltpu.VMEM((LANES,), jnp.int32),
            pltpu.VMEM((256,), jnp.int32),
            pltpu.VMEM((272,), jnp.int32),
            pltpu.VMEM((256,), jnp.int32),
            pltpu.VMEM((256,), jnp.int32),
            pltpu.VMEM((272,), jnp.int32),
            pltpu.VMEM((256,), jnp.int32),
            pltpu.VMEM((256,), jnp.int32),
            pltpu.VMEM((272,), jnp.int32),
            pltpu.VMEM((256,), jnp.int32),
            pltpu.VMEM((DEPTH, FACTORS, CW), jnp.float32),
            pltpu.VMEM((RING * FACTORS,), jnp.float32),
            pltpu.SemaphoreType.DMA,
            pltpu.SemaphoreType.DMA,
            pltpu.SemaphoreType.DMA,
            pltpu.SemaphoreType.DMA,
            pltpu.SemaphoreType.DMA,
            pltpu.SemaphoreType.DMA,
            pltpu.SemaphoreType.DMA,
        ],
    )


def _make_phase2():
    b_per_w = B // NUM_WORKERS
    mesh = plsc.VectorSubcoreMesh(core_axis_name="c", subcore_axis_name="s")
    return pl.kernel(
        _phase2_body,
        mesh=mesh,
        compiler_params=pltpu.CompilerParams(needs_layout_passes=False),
        out_type=jax.ShapeDtypeStruct((B,), jnp.float32),
        scratch_types=[
            pltpu.VMEM((b_per_w * FACTORS,), jnp.float32),
            pltpu.VMEM((b_per_w * FACTORS,), jnp.float32),
            pltpu.VMEM((b_per_w * FACTORS,), jnp.float32),
            pltpu.VMEM((b_per_w,), jnp.int32),
            pltpu.VMEM((b_per_w,), jnp.int32),
            pltpu.VMEM((b_per_w,), jnp.float32),
            pltpu.VMEM((b_per_w,), jnp.float32),
            pltpu.VMEM((b_per_w,), jnp.float32),
            pltpu.SemaphoreType.DMA,
        ],
    )


def kernel(user, item_i, item_j, user_table, item_table, item_bias_table):
    u32 = user.astype(jnp.int32)
    i32 = item_i.astype(jnp.int32)
    j32 = item_j.astype(jnp.int32)

    tt_u = user_table.T
    tt_i = item_table.T
    tl_u = jnp.pad(user_table[TAIL_BASE:].T,
                   ((0, 0), (0, CW - (V - TAIL_BASE))))
    tl_i = jnp.pad(item_table[TAIL_BASE:].T,
                   ((0, 0), (0, CW - (V - TAIL_BASE))))
    b1 = item_bias_table.reshape(-1)

    stg_u, stg_i, stg_j = _make_phase1()(
        tt_u, tt_i, tl_u, tl_i, u32, i32, j32)
    return _make_phase2()(stg_u, stg_i, stg_j, b1, i32, j32)
